# Initial kernel scaffold; baseline (speedup 1.0000x reference)
import jax
import jax.numpy as jnp
from jax.experimental import pallas as pl


def kernel(lab_codes, lab_values, code_table, value_W, value_b, out_W, out_b):
    raise NotImplementedError("write your pallas kernel here")


# speedup vs baseline: 12.2310x; 12.2310x over previous
import functools

import jax
import jax.numpy as jnp
from jax import lax
from jax.experimental import pallas as pl
from jax.experimental.pallas import tpu as pltpu
from jax.experimental.pallas import tpu_sc as plsc


def _sc_gather(idx2d, table, n_rows, half):
    info = plsc.get_sparse_core_info()
    nc, ns = info.num_cores, info.num_subcores
    nw = nc * ns
    rows_per_w = n_rows // nw
    KC = 20
    CK = KC * 128
    n_chunks = rows_per_w // CK
    assert rows_per_w % CK == 0

    idx_tiles = rows_per_w // 128

    mesh = plsc.VectorSubcoreMesh(core_axis_name="c", subcore_axis_name="s")

    @functools.partial(
        pl.kernel,
        out_type=jax.ShapeDtypeStruct((n_rows, half), jnp.float32),
        mesh=mesh,
        compiler_params=pltpu.CompilerParams(use_tc_tiling_on_sc=False),
        scratch_types=[
            pltpu.VMEM((idx_tiles, 128), jnp.int32),
            pltpu.VMEM((CK, half), jnp.float32),
            pltpu.SemaphoreType.DMA,
        ],
    )
    def gather_kernel(idx_hbm, table_hbm, out_hbm, idx_v, rows_v, sem):
        wid = lax.axis_index("s") * nc + lax.axis_index("c")
        row_base = wid * rows_per_w
        pltpu.sync_copy(idx_hbm.at[pl.ds(wid * idx_tiles, idx_tiles)], idx_v)

        def body(i, carry):
            copies = []
            for j in range(KC):
                copies.append(
                    pltpu.async_copy(
                        table_hbm.at[idx_v.at[i * KC + j]],
                        rows_v.at[pl.ds(j * 128, 128)],
                        sem,
                    )
                )
            for cpy in copies:
                cpy.wait()
            pltpu.sync_copy(rows_v, out_hbm.at[pl.ds(row_base + i * CK, CK)])
            return carry

        lax.fori_loop(0, n_chunks, body, 0)

    return gather_kernel(idx2d, table)


def _tc_fused(emb, vals2d, w1t, u_row, c_row, n_rows, half, hidden):
    BN = 2048
    grid = n_rows // BN

    def body(emb_ref, vals_ref, w_ref, u_ref, c_ref, out_ref):
        acc = jnp.dot(emb_ref[...], w_ref[...], preferred_element_type=jnp.float32)
        out_ref[...] = acc + vals_ref[...] * u_ref[...] + c_ref[...]

    return pl.pallas_call(
        body,
        grid=(grid,),
        in_specs=[
            pl.BlockSpec((BN, half), lambda i: (i, 0)),
            pl.BlockSpec((BN, 1), lambda i: (i, 0)),
            pl.BlockSpec((half, hidden), lambda i: (0, 0)),
            pl.BlockSpec((1, hidden), lambda i: (0, 0)),
            pl.BlockSpec((1, hidden), lambda i: (0, 0)),
        ],
        out_specs=pl.BlockSpec((BN, hidden), lambda i: (i, 0)),
        out_shape=jax.ShapeDtypeStruct((n_rows, hidden), jnp.float32),
    )(emb, vals2d, w1t, u_row, c_row)


def kernel(lab_codes, lab_values, code_table, value_W, value_b, out_W, out_b):
    B, L = lab_codes.shape
    vocab, half = code_table.shape
    hidden = out_W.shape[0]
    n_rows = B * L

    w1t = out_W[:, :half].T
    w2 = out_W[:, half:]
    u_row = (w2 @ value_W[:, 0]).reshape(1, hidden)
    c_row = (w2 @ value_b + out_b).reshape(1, hidden)

    idx2d = lab_codes.reshape(n_rows // 128, 128).astype(jnp.int32)
    emb = _sc_gather(idx2d, code_table, n_rows, half)

    vals2d = lab_values.reshape(n_rows, 1)
    out = _tc_fused(emb, vals2d, w1t, u_row, c_row, n_rows, half, hidden)
    return out.reshape(B, L, hidden)


# speedup vs baseline: 13.9180x; 1.1379x over previous
import functools

import jax
import jax.numpy as jnp
from jax import lax
from jax.experimental import pallas as pl
from jax.experimental.pallas import tpu as pltpu
from jax.experimental.pallas import tpu_sc as plsc


def _sc_gather(idx2d, table, n_rows, half):
    info = plsc.get_sparse_core_info()
    nc, ns = info.num_cores, info.num_subcores
    nw = nc * ns
    rows_per_w = n_rows // nw
    KC = 20
    CK = KC * 128
    n_chunks = rows_per_w // CK
    assert rows_per_w % CK == 0

    idx_tiles = rows_per_w // 128

    mesh = plsc.VectorSubcoreMesh(core_axis_name="c", subcore_axis_name="s")

    @functools.partial(
        pl.kernel,
        out_type=jax.ShapeDtypeStruct((n_rows, half), jnp.float32),
        mesh=mesh,
        compiler_params=pltpu.CompilerParams(use_tc_tiling_on_sc=False),
        scratch_types=[
            pltpu.VMEM((idx_tiles, 128), jnp.int32),
            pltpu.VMEM((CK, half), jnp.float32),
            pltpu.SemaphoreType.DMA,
        ],
    )
    def gather_kernel(idx_hbm, table_hbm, out_hbm, idx_v, rows_v, sem):
        wid = lax.axis_index("s") * nc + lax.axis_index("c")
        row_base = wid * rows_per_w
        pltpu.sync_copy(idx_hbm.at[pl.ds(wid * idx_tiles, idx_tiles)], idx_v)

        def body(i, carry):
            copies = []
            for j in range(KC):
                copies.append(
                    pltpu.async_copy(
                        table_hbm.at[idx_v.at[i * KC + j]],
                        rows_v.at[pl.ds(j * 128, 128)],
                        sem,
                    )
                )
            for cpy in copies:
                cpy.wait()
            pltpu.sync_copy(rows_v, out_hbm.at[pl.ds(row_base + i * CK, CK)])
            return carry

        lax.fori_loop(0, n_chunks, body, 0)

    return gather_kernel(idx2d, table)


def _tc_fused(emb, vals_rows, w1t, u_row, c_row, n_rows, half, hidden):
    BN = vals_rows.shape[2]
    grid = n_rows // BN

    def body(emb_ref, vals_ref, w_ref, u_ref, c_ref, out_ref):
        acc = jnp.dot(emb_ref[...], w_ref[...], preferred_element_type=jnp.float32)
        outer = jax.lax.dot_general(
            vals_ref[0], u_ref[...], (((0,), (0,)), ((), ())),
            preferred_element_type=jnp.float32,
        )
        out_ref[...] = acc + outer + c_ref[...]

    return pl.pallas_call(
        body,
        grid=(grid,),
        in_specs=[
            pl.BlockSpec((BN, half), lambda i: (i, 0)),
            pl.BlockSpec((1, 1, BN), lambda i: (i, 0, 0)),
            pl.BlockSpec((half, hidden), lambda i: (0, 0)),
            pl.BlockSpec((1, hidden), lambda i: (0, 0)),
            pl.BlockSpec((1, hidden), lambda i: (0, 0)),
        ],
        out_specs=pl.BlockSpec((BN, hidden), lambda i: (i, 0)),
        out_shape=jax.ShapeDtypeStruct((n_rows, hidden), jnp.float32),
    )(emb, vals_rows, w1t, u_row, c_row)


def kernel(lab_codes, lab_values, code_table, value_W, value_b, out_W, out_b):
    B, L = lab_codes.shape
    vocab, half = code_table.shape
    hidden = out_W.shape[0]
    n_rows = B * L

    w1t = out_W[:, :half].T
    w2 = out_W[:, half:]
    u_row = (w2 @ value_W[:, 0]).reshape(1, hidden)
    c_row = (w2 @ value_b + out_b).reshape(1, hidden)

    idx2d = lab_codes.reshape(n_rows // 128, 128).astype(jnp.int32)
    emb = _sc_gather(idx2d, code_table, n_rows, half)

    vals_rows = lab_values.reshape(n_rows // 2048, 1, 2048)
    out = _tc_fused(emb, vals_rows, w1t, u_row, c_row, n_rows, half, hidden)
    return out.reshape(B, L, hidden)
